# Initial kernel scaffold; baseline (speedup 1.0000x reference)
import jax
import jax.numpy as jnp
from jax.experimental import pallas as pl


def kernel(x, W_state, b_state, W_out, b_out, W_upd, b_upd):
    raise NotImplementedError("write your pallas kernel here")


# speedup vs baseline: 1.6248x; 1.6248x over previous
import functools

import jax
import jax.numpy as jnp
import numpy as np
from jax.experimental import pallas as pl
from jax.experimental.pallas import tpu as pltpu

_B, _T = 64, 128
_CB, _DB = 8, 256
_IN_DIM = _CB + _DB
_HS = 512
_H = 4
_NS = 3
_M = 128
_A = 2048
_PPH = _M + 1 + 1 + _NS + 1 + 2 * _M
_US = _H * _PPH
_EPS = 1e-8

_BB = 8


def _ntm_body(x_ref, wx_ref, wh_ref, wr_ref, bs_ref, wo_ref, bo_ref,
              wu_ref, bu_ref, out_ref, h_s, wt_s, mem_s,
              *, T, H, M, A, NS, eps):
    h_s[...] = jnp.ones_like(h_s)
    lane = jax.lax.broadcasted_iota(jnp.int32, wt_s.shape, 2)
    wt_s[...] = jnp.where(lane == 0, 1.0, 0.0).astype(jnp.float32)
    mem_s[...] = jnp.full_like(mem_s, 0.01)

    KE = H * M
    KA = 2 * H * M
    KS = 3 * H * M

    def step(t, carry):
        x_t = x_ref[t]
        mem = mem_s[...]
        h = h_s[...]

        reads = []
        for hh in range(H):
            w_h = wt_s[hh]
            r = jnp.sum(w_h[:, None, :] * mem, axis=2)
            reads.append(r)

        pre = jnp.dot(x_t, wx_ref[...], preferred_element_type=jnp.float32)
        pre = pre + jnp.dot(h, wh_ref[...], preferred_element_type=jnp.float32)
        for hh in range(H):
            pre = pre + jnp.dot(reads[hh], wr_ref[hh * M:(hh + 1) * M, :],
                                preferred_element_type=jnp.float32)
        h_new = jax.nn.sigmoid(pre + bs_ref[...])
        out = jax.nn.sigmoid(
            jnp.dot(h_new, wo_ref[...], preferred_element_type=jnp.float32)
            + bo_ref[...])
        upd = jnp.dot(h_new, wu_ref[...],
                      preferred_element_type=jnp.float32) + bu_ref[...]
        h_s[...] = h_new

        m_norm = jnp.sqrt(jnp.sum(mem * mem, axis=1))

        sc = upd[:, KS:KS + 6 * H]

        eacc = None
        aacc = None
        for hh in range(H):
            k_h = upd[:, hh * M:(hh + 1) * M]
            e_h = jax.nn.sigmoid(upd[:, KE + hh * M:KE + (hh + 1) * M])
            a_h = jnp.tanh(upd[:, KA + hh * M:KA + (hh + 1) * M])
            beta = jax.nn.softplus(sc[:, hh:hh + 1])
            g = jax.nn.sigmoid(sc[:, H + hh:H + hh + 1])
            sh_raw = sc[:, 2 * H + NS * hh:2 * H + NS * (hh + 1)]
            gamma = 1.0 + jax.nn.softplus(sc[:, 5 * H + hh:5 * H + hh + 1])

            dots = jnp.sum(k_h[:, :, None] * mem, axis=1)
            k_norm = jnp.sqrt(jnp.sum(k_h * k_h, axis=1, keepdims=True))
            sim = dots / (k_norm * m_norm + eps)
            z = beta * sim
            z = z - jnp.max(z, axis=1, keepdims=True)
            ez = jnp.exp(z)
            wc = ez / jnp.sum(ez, axis=1, keepdims=True)

            wg = g * wc + (1.0 - g) * wt_s[hh]

            sh = jax.nn.softmax(sh_raw, axis=-1)
            ws = (sh[:, 0:1] * jnp.roll(wg, -1, axis=1)
                  + sh[:, 1:2] * wg
                  + sh[:, 2:3] * jnp.roll(wg, 1, axis=1))

            wp = jnp.exp(gamma * jnp.log(ws + eps))
            w_new = wp / jnp.sum(wp, axis=1, keepdims=True)
            wt_s[hh] = w_new

            term = 1.0 - e_h[:, :, None] * w_new[:, None, :]
            eacc = term if eacc is None else eacc * term
            at = a_h[:, :, None] * w_new[:, None, :]
            aacc = at if aacc is None else aacc + at

        mem_s[...] = mem * eacc + aacc
        out_ref[t] = out
        return carry

    jax.lax.fori_loop(0, T, step, 0)


def kernel(x, W_state, b_state, W_out, b_out, W_upd, b_upd):
    Wx = W_state[:_IN_DIM]
    Wh = W_state[_IN_DIM:_IN_DIM + _HS]
    Wr = W_state[_IN_DIM + _HS:]

    perm = np.empty((_US,), np.int32)
    p = 0
    for hh in range(_H):
        perm[p:p + _M] = hh * _PPH + np.arange(_M); p += _M
    for hh in range(_H):
        perm[p:p + _M] = hh * _PPH + _M + 2 + _NS + 1 + np.arange(_M); p += _M
    for hh in range(_H):
        perm[p:p + _M] = hh * _PPH + 2 * _M + 3 + _NS + np.arange(_M); p += _M
    for hh in range(_H):
        perm[p] = hh * _PPH + _M; p += 1
    for hh in range(_H):
        perm[p] = hh * _PPH + _M + 1; p += 1
    for hh in range(_H):
        perm[p:p + _NS] = hh * _PPH + _M + 2 + np.arange(_NS); p += _NS
    for hh in range(_H):
        perm[p] = hh * _PPH + _M + 2 + _NS; p += 1
    Wu = W_upd[:, perm]
    bu = b_upd[perm].reshape(1, _US)

    xt = jnp.swapaxes(x, 0, 1)
    bs = b_state.reshape(1, _HS)
    bo = b_out.reshape(1, _DB)

    nb = _B // _BB
    body = functools.partial(_ntm_body, T=_T, H=_H, M=_M, A=_A, NS=_NS,
                             eps=_EPS)
    out_t = pl.pallas_call(
        body,
        grid=(nb,),
        in_specs=[
            pl.BlockSpec((_T, _BB, _IN_DIM), lambda i: (0, i, 0)),
            pl.BlockSpec((_IN_DIM, _HS), lambda i: (0, 0)),
            pl.BlockSpec((_HS, _HS), lambda i: (0, 0)),
            pl.BlockSpec((_H * _M, _HS), lambda i: (0, 0)),
            pl.BlockSpec((1, _HS), lambda i: (0, 0)),
            pl.BlockSpec((_HS, _DB), lambda i: (0, 0)),
            pl.BlockSpec((1, _DB), lambda i: (0, 0)),
            pl.BlockSpec((_HS, _US), lambda i: (0, 0)),
            pl.BlockSpec((1, _US), lambda i: (0, 0)),
        ],
        out_specs=pl.BlockSpec((_T, _BB, _DB), lambda i: (0, i, 0)),
        out_shape=jax.ShapeDtypeStruct((_T, _B, _DB), jnp.float32),
        scratch_shapes=[
            pltpu.VMEM((_BB, _HS), jnp.float32),
            pltpu.VMEM((_H, _BB, _A), jnp.float32),
            pltpu.VMEM((_BB, _M, _A), jnp.float32),
        ],
        compiler_params=pltpu.CompilerParams(
            dimension_semantics=("parallel",),
        ),
    )(xt, Wx, Wh, Wr, bs, W_out, bo, Wu, bu)
    return jnp.swapaxes(out_t, 0, 1)


# speedup vs baseline: 2.8752x; 1.7695x over previous
import functools

import jax
import jax.numpy as jnp
import numpy as np
from jax.experimental import pallas as pl
from jax.experimental.pallas import tpu as pltpu

_B, _T = 64, 128
_CB, _DB = 8, 256
_IN_DIM = _CB + _DB
_HS = 512
_H = 4
_NS = 3
_M = 128
_A = 2048
_PPH = _M + 1 + 1 + _NS + 1 + 2 * _M
_US = _H * _PPH
_EPS = 1e-8

_BB = 8


def _ntm_body(x_ref, wx_ref, wh_ref, wr_ref, bs_ref, wo_ref, bo_ref,
              wu_ref, bu_ref, out_ref, h_s, wt_s, mem_s, mn_s,
              *, T, H, M, A, NS, eps):
    BB = h_s.shape[0]
    h_s[...] = jnp.ones_like(h_s)
    lane = jax.lax.broadcasted_iota(jnp.int32, wt_s.shape, 2)
    wt_s[...] = jnp.where(lane == 0, 1.0, 0.0).astype(jnp.float32)
    mem_s[...] = jnp.full_like(mem_s, 0.01)
    mn_s[...] = jnp.full_like(mn_s, float(np.sqrt(M) * 0.01))

    KE = H * M
    KA = 2 * H * M
    KS = 3 * H * M

    def step(t, carry):
        x_t = x_ref[t]
        h = h_s[...]

        R = jnp.stack(
            [jax.lax.dot_general(wt_s[b], mem_s[b], (((1,), (1,)), ((), ())),
                                 preferred_element_type=jnp.float32)
             for b in range(BB)], axis=0)
        read_flat = R.reshape(BB, H * M)

        pre = jnp.dot(x_t, wx_ref[...], preferred_element_type=jnp.float32)
        pre = pre + jnp.dot(h, wh_ref[...], preferred_element_type=jnp.float32)
        pre = pre + jnp.dot(read_flat, wr_ref[...],
                            preferred_element_type=jnp.float32)
        h_new = jax.nn.sigmoid(pre + bs_ref[...])
        h_s[...] = h_new
        out = jax.nn.sigmoid(
            jnp.dot(h_new, wo_ref[...], preferred_element_type=jnp.float32)
            + bo_ref[...])
        out_ref[t] = out
        upd = jnp.dot(h_new, wu_ref[...],
                      preferred_element_type=jnp.float32) + bu_ref[...]

        kb = upd[:, :KE].reshape(BB, H, M)
        eb = jax.nn.sigmoid(upd[:, KE:KA]).reshape(BB, H, M)
        ab = jnp.tanh(upd[:, KA:KS]).reshape(BB, H, M)
        sc = upd[:, KS:KS + 6 * H]
        beta3 = jax.nn.softplus(sc[:, :H]).reshape(BB, H, 1)
        g3 = jax.nn.sigmoid(sc[:, H:2 * H]).reshape(BB, H, 1)
        sh3 = jax.nn.softmax(sc[:, 2 * H:2 * H + NS * H].reshape(BB, H, NS),
                             axis=2)
        gamma3 = 1.0 + jax.nn.softplus(sc[:, 5 * H:6 * H]).reshape(BB, H, 1)

        D = jnp.stack(
            [jnp.dot(kb[b], mem_s[b], preferred_element_type=jnp.float32)
             for b in range(BB)], axis=0)
        k_norm = jnp.sqrt(jnp.sum(kb * kb, axis=2, keepdims=True))
        sim = D / (k_norm * mn_s[...][:, None, :] + eps)
        z = beta3 * sim
        z = z - jnp.max(z, axis=2, keepdims=True)
        ez = jnp.exp(z)
        wc = ez / jnp.sum(ez, axis=2, keepdims=True)

        wg = g3 * wc + (1.0 - g3) * wt_s[...]

        ws = (sh3[:, :, 0:1] * jnp.roll(wg, -1, axis=2)
              + sh3[:, :, 1:2] * wg
              + sh3[:, :, 2:3] * jnp.roll(wg, 1, axis=2))

        wp = jnp.exp(gamma3 * jnp.log(ws + eps))
        wtn = wp / jnp.sum(wp, axis=2, keepdims=True)
        wt_s[...] = wtn

        AT = jnp.stack(
            [jax.lax.dot_general(ab[b], wtn[b], (((0,), (0,)), ((), ())),
                                 preferred_element_type=jnp.float32)
             for b in range(BB)], axis=0)
        eacc = None
        for hh in range(H):
            term = 1.0 - eb[:, hh, :][:, :, None] * wtn[:, hh, :][:, None, :]
            eacc = term if eacc is None else eacc * term
        mem_new = mem_s[...] * eacc + AT
        mem_s[...] = mem_new
        mn_s[...] = jnp.sqrt(jnp.sum(mem_new * mem_new, axis=1))
        return carry

    jax.lax.fori_loop(0, T, step, 0)


def kernel(x, W_state, b_state, W_out, b_out, W_upd, b_upd):
    Wx = W_state[:_IN_DIM]
    Wh = W_state[_IN_DIM:_IN_DIM + _HS]
    Wr = W_state[_IN_DIM + _HS:]

    perm = np.empty((_US,), np.int32)
    p = 0
    for hh in range(_H):
        perm[p:p + _M] = hh * _PPH + np.arange(_M); p += _M
    for hh in range(_H):
        perm[p:p + _M] = hh * _PPH + _M + 2 + _NS + 1 + np.arange(_M); p += _M
    for hh in range(_H):
        perm[p:p + _M] = hh * _PPH + 2 * _M + 3 + _NS + np.arange(_M); p += _M
    for hh in range(_H):
        perm[p] = hh * _PPH + _M; p += 1
    for hh in range(_H):
        perm[p] = hh * _PPH + _M + 1; p += 1
    for hh in range(_H):
        perm[p:p + _NS] = hh * _PPH + _M + 2 + np.arange(_NS); p += _NS
    for hh in range(_H):
        perm[p] = hh * _PPH + _M + 2 + _NS; p += 1
    Wu = W_upd[:, perm]
    bu = b_upd[perm].reshape(1, _US)

    xt = jnp.swapaxes(x, 0, 1)
    bs = b_state.reshape(1, _HS)
    bo = b_out.reshape(1, _DB)

    nb = _B // _BB
    body = functools.partial(_ntm_body, T=_T, H=_H, M=_M, A=_A, NS=_NS,
                             eps=_EPS)
    out_t = pl.pallas_call(
        body,
        grid=(nb,),
        in_specs=[
            pl.BlockSpec((_T, _BB, _IN_DIM), lambda i: (0, i, 0)),
            pl.BlockSpec((_IN_DIM, _HS), lambda i: (0, 0)),
            pl.BlockSpec((_HS, _HS), lambda i: (0, 0)),
            pl.BlockSpec((_H * _M, _HS), lambda i: (0, 0)),
            pl.BlockSpec((1, _HS), lambda i: (0, 0)),
            pl.BlockSpec((_HS, _DB), lambda i: (0, 0)),
            pl.BlockSpec((1, _DB), lambda i: (0, 0)),
            pl.BlockSpec((_HS, _US), lambda i: (0, 0)),
            pl.BlockSpec((1, _US), lambda i: (0, 0)),
        ],
        out_specs=pl.BlockSpec((_T, _BB, _DB), lambda i: (0, i, 0)),
        out_shape=jax.ShapeDtypeStruct((_T, _B, _DB), jnp.float32),
        scratch_shapes=[
            pltpu.VMEM((_BB, _HS), jnp.float32),
            pltpu.VMEM((_BB, _H, _A), jnp.float32),
            pltpu.VMEM((_BB, _M, _A), jnp.float32),
            pltpu.VMEM((_BB, _A), jnp.float32),
        ],
        compiler_params=pltpu.CompilerParams(
            dimension_semantics=("parallel",),
        ),
    )(xt, Wx, Wh, Wr, bs, W_out, bo, Wu, bu)
    return jnp.swapaxes(out_t, 0, 1)


# speedup vs baseline: 3.9927x; 1.3887x over previous
import functools

import jax
import jax.numpy as jnp
import numpy as np
from jax.experimental import pallas as pl
from jax.experimental.pallas import tpu as pltpu

_B, _T = 64, 128
_CB, _DB = 8, 256
_IN_DIM = _CB + _DB
_HS = 512
_H = 4
_NS = 3
_M = 128
_A = 2048
_PPH = _M + 1 + 1 + _NS + 1 + 2 * _M
_US = _H * _PPH
_EPS = 1e-8

_BB = 8


def _ntm_body(x_ref, wx_ref, wh_ref, wr_ref, bs_ref, wo_ref, bo_ref,
              wu_ref, bu_ref, out_ref, h_s, wt_s, mem_s, mn_s,
              *, T, H, M, A, NS, eps):
    BB = h_s.shape[0]
    h_s[...] = jnp.ones_like(h_s)
    lane = jax.lax.broadcasted_iota(jnp.int32, wt_s.shape, 2)
    wt_s[...] = jnp.where(lane == 0, 1.0, 0.0).astype(jnp.float32)
    mem_s[...] = jnp.full_like(mem_s, 0.01)
    mn_s[...] = jnp.full_like(mn_s, float(np.sqrt(M) * 0.01))

    KE = H * M
    KA = 2 * H * M
    KS = 3 * H * M

    def step(t, carry):
        x_t = x_ref[t]
        h = h_s[...]

        R = jnp.stack(
            [jax.lax.dot_general(wt_s[b], mem_s[b], (((1,), (1,)), ((), ())),
                                 preferred_element_type=jnp.float32)
             for b in range(BB)], axis=0)
        read_flat = R.reshape(BB, H * M)

        pre = jnp.dot(x_t, wx_ref[...], preferred_element_type=jnp.float32)
        pre = pre + jnp.dot(h, wh_ref[...], preferred_element_type=jnp.float32)
        pre = pre + jnp.dot(read_flat, wr_ref[...],
                            preferred_element_type=jnp.float32)
        h_new = jax.nn.sigmoid(pre + bs_ref[...])
        h_s[...] = h_new
        out = jax.nn.sigmoid(
            jnp.dot(h_new, wo_ref[...], preferred_element_type=jnp.float32)
            + bo_ref[...])
        out_ref[t] = out
        upd = jnp.dot(h_new, wu_ref[...],
                      preferred_element_type=jnp.float32) + bu_ref[...]

        kb = upd[:, :KE].reshape(BB, H, M)
        eb = jax.nn.sigmoid(upd[:, KE:KA]).reshape(BB, H, M)
        ab = jnp.tanh(upd[:, KA:KS]).reshape(BB, H, M)
        sc = upd[:, KS:KS + 6 * H]
        beta3 = jax.nn.softplus(sc[:, :H]).reshape(BB, H, 1)
        g3 = jax.nn.sigmoid(sc[:, H:2 * H]).reshape(BB, H, 1)
        sh3 = jax.nn.softmax(sc[:, 2 * H:2 * H + NS * H].reshape(BB, H, NS),
                             axis=2)
        gamma3 = 1.0 + jax.nn.softplus(sc[:, 5 * H:6 * H]).reshape(BB, H, 1)

        D = jnp.stack(
            [jnp.dot(kb[b], mem_s[b], preferred_element_type=jnp.float32)
             for b in range(BB)], axis=0)
        k_norm = jnp.sqrt(jnp.sum(kb * kb, axis=2, keepdims=True))
        sim = D / (k_norm * mn_s[...][:, None, :] + eps)
        z = beta3 * sim
        z = z - jnp.max(z, axis=2, keepdims=True)
        ez = jnp.exp(z)
        wc = ez / jnp.sum(ez, axis=2, keepdims=True)

        wg = g3 * wc + (1.0 - g3) * wt_s[...]

        ws = (sh3[:, :, 0:1] * jnp.roll(wg, -1, axis=2)
              + sh3[:, :, 1:2] * wg
              + sh3[:, :, 2:3] * jnp.roll(wg, 1, axis=2))

        wp = jnp.exp(gamma3 * jnp.log(ws + eps))
        wtn = wp / jnp.sum(wp, axis=2, keepdims=True)
        wt_s[...] = wtn

        AT = jnp.stack(
            [jax.lax.dot_general(ab[b], wtn[b], (((0,), (0,)), ((), ())),
                                 preferred_element_type=jnp.float32)
             for b in range(BB)], axis=0)
        ones_m = jnp.ones((BB, M), jnp.float32)
        ones_a = jnp.ones((BB, A), jnp.float32)
        E_list, W_list = [ones_m], [ones_a]
        for hh in range(H):
            neg_e = -eb[:, hh, :]
            w_h = wtn[:, hh, :]
            E_list = E_list + [ev * neg_e for ev in E_list]
            W_list = W_list + [wv * w_h for wv in W_list]
        E_ext = jnp.stack(E_list, axis=2)
        W_ext = jnp.stack(W_list, axis=1)
        eacc = jnp.stack(
            [jnp.dot(E_ext[b], W_ext[b], preferred_element_type=jnp.float32)
             for b in range(BB)], axis=0)
        mem_new = mem_s[...] * eacc + AT
        mem_s[...] = mem_new
        mn_s[...] = jnp.sqrt(jnp.sum(mem_new * mem_new, axis=1))
        return carry

    jax.lax.fori_loop(0, T, step, 0)


def kernel(x, W_state, b_state, W_out, b_out, W_upd, b_upd):
    Wx = W_state[:_IN_DIM]
    Wh = W_state[_IN_DIM:_IN_DIM + _HS]
    Wr = W_state[_IN_DIM + _HS:]

    perm = np.empty((_US,), np.int32)
    p = 0
    for hh in range(_H):
        perm[p:p + _M] = hh * _PPH + np.arange(_M); p += _M
    for hh in range(_H):
        perm[p:p + _M] = hh * _PPH + _M + 2 + _NS + 1 + np.arange(_M); p += _M
    for hh in range(_H):
        perm[p:p + _M] = hh * _PPH + 2 * _M + 3 + _NS + np.arange(_M); p += _M
    for hh in range(_H):
        perm[p] = hh * _PPH + _M; p += 1
    for hh in range(_H):
        perm[p] = hh * _PPH + _M + 1; p += 1
    for hh in range(_H):
        perm[p:p + _NS] = hh * _PPH + _M + 2 + np.arange(_NS); p += _NS
    for hh in range(_H):
        perm[p] = hh * _PPH + _M + 2 + _NS; p += 1
    Wu = W_upd[:, perm]
    bu = b_upd[perm].reshape(1, _US)

    xt = jnp.swapaxes(x, 0, 1)
    bs = b_state.reshape(1, _HS)
    bo = b_out.reshape(1, _DB)

    nb = _B // _BB
    body = functools.partial(_ntm_body, T=_T, H=_H, M=_M, A=_A, NS=_NS,
                             eps=_EPS)
    out_t = pl.pallas_call(
        body,
        grid=(nb,),
        in_specs=[
            pl.BlockSpec((_T, _BB, _IN_DIM), lambda i: (0, i, 0)),
            pl.BlockSpec((_IN_DIM, _HS), lambda i: (0, 0)),
            pl.BlockSpec((_HS, _HS), lambda i: (0, 0)),
            pl.BlockSpec((_H * _M, _HS), lambda i: (0, 0)),
            pl.BlockSpec((1, _HS), lambda i: (0, 0)),
            pl.BlockSpec((_HS, _DB), lambda i: (0, 0)),
            pl.BlockSpec((1, _DB), lambda i: (0, 0)),
            pl.BlockSpec((_HS, _US), lambda i: (0, 0)),
            pl.BlockSpec((1, _US), lambda i: (0, 0)),
        ],
        out_specs=pl.BlockSpec((_T, _BB, _DB), lambda i: (0, i, 0)),
        out_shape=jax.ShapeDtypeStruct((_T, _B, _DB), jnp.float32),
        scratch_shapes=[
            pltpu.VMEM((_BB, _HS), jnp.float32),
            pltpu.VMEM((_BB, _H, _A), jnp.float32),
            pltpu.VMEM((_BB, _M, _A), jnp.float32),
            pltpu.VMEM((_BB, _A), jnp.float32),
        ],
        compiler_params=pltpu.CompilerParams(
            dimension_semantics=("parallel",),
        ),
    )(xt, Wx, Wh, Wr, bs, W_out, bo, Wu, bu)
    return jnp.swapaxes(out_t, 0, 1)


# speedup vs baseline: 4.2092x; 1.0542x over previous
import functools

import jax
import jax.numpy as jnp
import numpy as np
from jax.experimental import pallas as pl
from jax.experimental.pallas import tpu as pltpu

_B, _T = 64, 128
_CB, _DB = 8, 256
_IN_DIM = _CB + _DB
_HS = 512
_H = 4
_NS = 3
_M = 128
_A = 2048
_PPH = _M + 1 + 1 + _NS + 1 + 2 * _M
_US = _H * _PPH
_EPS = 1e-8

_BB = 16


def _ntm_body(x_ref, wx_ref, wh_ref, wr_ref, bs_ref, wo_ref, bo_ref,
              wu_ref, bu_ref, out_ref, h_s, wt_s, mem_s, mn_s,
              *, T, H, M, A, NS, eps):
    BB = h_s.shape[0]
    h_s[...] = jnp.ones_like(h_s)
    lane = jax.lax.broadcasted_iota(jnp.int32, wt_s.shape, 2)
    wt_s[...] = jnp.where(lane == 0, 1.0, 0.0).astype(jnp.float32)
    mem_s[...] = jnp.full_like(mem_s, 0.01)
    mn_s[...] = jnp.full_like(mn_s, float(np.sqrt(M) * 0.01))

    KE = H * M
    KA = 2 * H * M
    KS = 3 * H * M

    def step(t, carry):
        x_t = x_ref[t]
        h = h_s[...]

        R = jnp.stack(
            [jax.lax.dot_general(wt_s[b], mem_s[b], (((1,), (1,)), ((), ())),
                                 preferred_element_type=jnp.float32)
             for b in range(BB)], axis=0)
        read_flat = R.reshape(BB, H * M)

        pre = jnp.dot(x_t, wx_ref[...], preferred_element_type=jnp.float32)
        pre = pre + jnp.dot(h, wh_ref[...], preferred_element_type=jnp.float32)
        pre = pre + jnp.dot(read_flat, wr_ref[...],
                            preferred_element_type=jnp.float32)
        h_new = jax.nn.sigmoid(pre + bs_ref[...])
        h_s[...] = h_new
        out = jax.nn.sigmoid(
            jnp.dot(h_new, wo_ref[...], preferred_element_type=jnp.float32)
            + bo_ref[...])
        out_ref[t] = out
        upd = jnp.dot(h_new, wu_ref[...],
                      preferred_element_type=jnp.float32) + bu_ref[...]

        kb = upd[:, :KE].reshape(BB, H, M)
        eb = jax.nn.sigmoid(upd[:, KE:KA]).reshape(BB, H, M)
        ab = jnp.tanh(upd[:, KA:KS]).reshape(BB, H, M)
        sc = upd[:, KS:KS + 6 * H]
        beta3 = jax.nn.softplus(sc[:, :H]).reshape(BB, H, 1)
        g3 = jax.nn.sigmoid(sc[:, H:2 * H]).reshape(BB, H, 1)
        sh3 = jax.nn.softmax(sc[:, 2 * H:2 * H + NS * H].reshape(BB, H, NS),
                             axis=2)
        gamma3 = 1.0 + jax.nn.softplus(sc[:, 5 * H:6 * H]).reshape(BB, H, 1)

        D = jnp.stack(
            [jnp.dot(kb[b], mem_s[b], preferred_element_type=jnp.float32)
             for b in range(BB)], axis=0)
        k_norm = jnp.sqrt(jnp.sum(kb * kb, axis=2, keepdims=True))
        sim = D / (k_norm * mn_s[...][:, None, :] + eps)
        z = beta3 * sim
        z = z - jnp.max(z, axis=2, keepdims=True)
        ez = jnp.exp(z)
        wc = ez / jnp.sum(ez, axis=2, keepdims=True)

        wg = g3 * wc + (1.0 - g3) * wt_s[...]

        ws = (sh3[:, :, 0:1] * jnp.roll(wg, -1, axis=2)
              + sh3[:, :, 1:2] * wg
              + sh3[:, :, 2:3] * jnp.roll(wg, 1, axis=2))

        wp = jnp.exp(gamma3 * jnp.log(ws + eps))
        wtn = wp / jnp.sum(wp, axis=2, keepdims=True)
        wt_s[...] = wtn

        AT = jnp.stack(
            [jax.lax.dot_general(ab[b], wtn[b], (((0,), (0,)), ((), ())),
                                 preferred_element_type=jnp.float32)
             for b in range(BB)], axis=0)
        ones_m = jnp.ones((BB, M), jnp.float32)
        ones_a = jnp.ones((BB, A), jnp.float32)
        E_list, W_list = [ones_m], [ones_a]
        for hh in range(H):
            neg_e = -eb[:, hh, :]
            w_h = wtn[:, hh, :]
            E_list = E_list + [ev * neg_e for ev in E_list]
            W_list = W_list + [wv * w_h for wv in W_list]
        E_ext = jnp.stack(E_list, axis=2)
        W_ext = jnp.stack(W_list, axis=1)
        eacc = jnp.stack(
            [jnp.dot(E_ext[b], W_ext[b], preferred_element_type=jnp.float32)
             for b in range(BB)], axis=0)
        mem_new = mem_s[...] * eacc + AT
        mem_s[...] = mem_new
        mn_s[...] = jnp.sqrt(jnp.sum(mem_new * mem_new, axis=1))
        return carry

    jax.lax.fori_loop(0, T, step, 0)


def kernel(x, W_state, b_state, W_out, b_out, W_upd, b_upd):
    Wx = W_state[:_IN_DIM]
    Wh = W_state[_IN_DIM:_IN_DIM + _HS]
    Wr = W_state[_IN_DIM + _HS:]

    perm = np.empty((_US,), np.int32)
    p = 0
    for hh in range(_H):
        perm[p:p + _M] = hh * _PPH + np.arange(_M); p += _M
    for hh in range(_H):
        perm[p:p + _M] = hh * _PPH + _M + 2 + _NS + 1 + np.arange(_M); p += _M
    for hh in range(_H):
        perm[p:p + _M] = hh * _PPH + 2 * _M + 3 + _NS + np.arange(_M); p += _M
    for hh in range(_H):
        perm[p] = hh * _PPH + _M; p += 1
    for hh in range(_H):
        perm[p] = hh * _PPH + _M + 1; p += 1
    for hh in range(_H):
        perm[p:p + _NS] = hh * _PPH + _M + 2 + np.arange(_NS); p += _NS
    for hh in range(_H):
        perm[p] = hh * _PPH + _M + 2 + _NS; p += 1
    Wu = W_upd[:, perm]
    bu = b_upd[perm].reshape(1, _US)

    xt = jnp.swapaxes(x, 0, 1)
    bs = b_state.reshape(1, _HS)
    bo = b_out.reshape(1, _DB)

    nb = _B // _BB
    body = functools.partial(_ntm_body, T=_T, H=_H, M=_M, A=_A, NS=_NS,
                             eps=_EPS)
    out_t = pl.pallas_call(
        body,
        grid=(nb,),
        in_specs=[
            pl.BlockSpec((_T, _BB, _IN_DIM), lambda i: (0, i, 0)),
            pl.BlockSpec((_IN_DIM, _HS), lambda i: (0, 0)),
            pl.BlockSpec((_HS, _HS), lambda i: (0, 0)),
            pl.BlockSpec((_H * _M, _HS), lambda i: (0, 0)),
            pl.BlockSpec((1, _HS), lambda i: (0, 0)),
            pl.BlockSpec((_HS, _DB), lambda i: (0, 0)),
            pl.BlockSpec((1, _DB), lambda i: (0, 0)),
            pl.BlockSpec((_HS, _US), lambda i: (0, 0)),
            pl.BlockSpec((1, _US), lambda i: (0, 0)),
        ],
        out_specs=pl.BlockSpec((_T, _BB, _DB), lambda i: (0, i, 0)),
        out_shape=jax.ShapeDtypeStruct((_T, _B, _DB), jnp.float32),
        scratch_shapes=[
            pltpu.VMEM((_BB, _HS), jnp.float32),
            pltpu.VMEM((_BB, _H, _A), jnp.float32),
            pltpu.VMEM((_BB, _M, _A), jnp.float32),
            pltpu.VMEM((_BB, _A), jnp.float32),
        ],
        compiler_params=pltpu.CompilerParams(
            dimension_semantics=("parallel",),
        ),
    )(xt, Wx, Wh, Wr, bs, W_out, bo, Wu, bu)
    return jnp.swapaxes(out_t, 0, 1)
